# baseline (speedup 1.0000x reference)
import functools

import jax
import jax.numpy as jnp
from jax import lax
from jax.experimental import pallas as pl
from jax.experimental.pallas import tpu as pltpu
from jax.experimental.pallas import tpu_sc as plsc

_BETA = 0.25

_R = 1024
_C = 1024


def _argmin_body(z_ref, cb_ref, z2_ref, e2_ref, idx_out_ref, loss_out_ref,
                 run_min_ref, run_idx_ref):
    j = pl.program_id(1)
    ncode = pl.num_programs(1)

    z = z_ref[...]
    cb = cb_ref[...]
    dots = lax.dot_general(z, cb, (((1,), (1,)), ((), ())),
                           preferred_element_type=jnp.float32)
    dist = (z2_ref[...] + e2_ref[...]) - 2.0 * dots

    tile_min = jnp.min(dist, axis=1, keepdims=True)
    code_ids = lax.broadcasted_iota(jnp.int32, dist.shape, 1) + j * _C
    cand = jnp.where(dist == tile_min, code_ids, jnp.int32(2**30))
    tile_arg = jnp.min(cand, axis=1, keepdims=True)

    @pl.when(j == 0)
    def _init():
        run_min_ref[...] = tile_min
        run_idx_ref[...] = tile_arg

    @pl.when(j > 0)
    def _acc():
        better = tile_min < run_min_ref[...]
        run_idx_ref[...] = jnp.where(better, tile_arg, run_idx_ref[...])
        run_min_ref[...] = jnp.where(better, tile_min, run_min_ref[...])

    @pl.when(j == ncode - 1)
    def _emit():
        idx_out_ref[...] = run_idx_ref[...]
        loss_out_ref[...] = jnp.reshape(jnp.sum(run_min_ref[...]), (1, 1, 1))


def _distance_argmin(z_flat, codebook, z2, e2):
    n, d = z_flat.shape
    ncodes = codebook.shape[0]
    nr, nc = n // _R, ncodes // _C
    idx2, loss_parts = pl.pallas_call(
        _argmin_body,
        grid=(nr, nc),
        in_specs=[
            pl.BlockSpec((_R, d), lambda i, j: (i, 0)),
            pl.BlockSpec((_C, d), lambda i, j: (j, 0)),
            pl.BlockSpec((_R, 1), lambda i, j: (i, 0)),
            pl.BlockSpec((1, _C), lambda i, j: (0, j)),
        ],
        out_specs=[
            pl.BlockSpec((_R, 1), lambda i, j: (i, 0)),
            pl.BlockSpec((1, 1, 1), lambda i, j: (i, 0, 0)),
        ],
        out_shape=[
            jax.ShapeDtypeStruct((n, 1), jnp.int32),
            jax.ShapeDtypeStruct((nr, 1, 1), jnp.float32),
        ],
        scratch_shapes=[
            pltpu.VMEM((_R, 1), jnp.float32),
            pltpu.VMEM((_R, 1), jnp.int32),
        ],
        compiler_params=pltpu.CompilerParams(
            dimension_semantics=("parallel", "arbitrary")),
    )(z_flat, codebook, z2, e2)
    return idx2.reshape(n), loss_parts


def _make_sc_gather(num_codes, dpad, n):
    info = plsc.get_sparse_core_info()
    ncore, nsub = info.num_cores, info.num_subcores
    nw = ncore * nsub
    rows_per_w = n // nw
    chunks = rows_per_w // 128
    mesh = plsc.VectorSubcoreMesh(core_axis_name="c", subcore_axis_name="s")

    @functools.partial(
        pl.kernel,
        mesh=mesh,
        out_type=jax.ShapeDtypeStruct((n, dpad), jnp.float32),
        scratch_types=[
            pltpu.VMEM((chunks, 128), jnp.int32),
            pltpu.VMEM((rows_per_w, dpad), jnp.float32),
            pltpu.SemaphoreType.DMA,
        ],
    )
    def gather_k(table_hbm, idx_hbm, out_hbm, idx_v, rows_v, sem):
        wid = lax.axis_index("s") * ncore + lax.axis_index("c")
        pltpu.sync_copy(idx_hbm.at[pl.ds(wid * chunks, chunks)], idx_v)
        cps = [
            pltpu.async_copy(table_hbm.at[idx_v.at[c]],
                             rows_v.at[pl.ds(c * 128, 128)], sem)
            for c in range(chunks)
        ]
        for cp in cps:
            cp.wait()
        pltpu.sync_copy(rows_v, out_hbm.at[pl.ds(wid * rows_per_w,
                                                 rows_per_w)])

    return gather_k


def kernel(z_e, codebook):
    b, d, h, w = z_e.shape
    n = b * h * w
    num_codes = codebook.shape[0]
    z_flat = jnp.transpose(z_e, (0, 2, 3, 1)).reshape(n, d)
    z2 = (z_flat ** 2).sum(axis=1, keepdims=True)
    e2 = (codebook ** 2).sum(axis=1)[None, :]

    indices, loss_parts = _distance_argmin(z_flat, codebook, z2, e2)

    idx2d = indices.reshape(-1, 128)
    cb_pad = jnp.pad(codebook, ((0, 0), (0, 128 - d)))
    zq_flat = _make_sc_gather(num_codes, 128, n)(cb_pad, idx2d)[:, :d]

    z_q = zq_flat.reshape(b, h, w, d).transpose(0, 3, 1, 2)
    vq_loss = (1.0 + _BETA) * (jnp.sum(loss_parts) / (n * d))
    indices_map = indices.reshape(b, h, w)
    z_q_st = z_e + (z_q - z_e)
    return (z_q_st, indices_map, vq_loss)
